# baseline (speedup 1.0000x reference)
import functools

import jax
import jax.numpy as jnp
from jax import lax
from jax.experimental import pallas as pl
from jax.experimental.pallas import tpu as pltpu
from jax.experimental.pallas import tpu_sc as plsc

_DIM = 64
_LANES = 128
_N_WORKERS = 32


def _gather_kernel(ids_hbm, w_hbm, out_hbm, idx_v, rows_v, sem, *,
                   rows_per_w):
    wid = lax.axis_index("s") * 2 + lax.axis_index("c")
    row0 = wid * rows_per_w
    pltpu.sync_copy(ids_hbm.at[wid], idx_v)

    def body(j, carry):
        pltpu.async_copy(w_hbm.at[idx_v.at[j]], rows_v, sem).wait()
        pltpu.sync_copy(rows_v, out_hbm.at[pl.ds((row0 + j) * _LANES, _LANES)])
        return carry

    lax.fori_loop(0, rows_per_w, body, 0)


def kernel(input_ids, weight):
    b, s = input_ids.shape
    total = b * s
    n_rows = total // _LANES
    rows_per_w = n_rows // _N_WORKERS

    ids3d = input_ids.reshape(_N_WORKERS, rows_per_w, _LANES).astype(jnp.int32)
    mesh = plsc.VectorSubcoreMesh(core_axis_name="c", subcore_axis_name="s")

    run = functools.partial(
        pl.kernel,
        mesh=mesh,
        out_type=jax.ShapeDtypeStruct((total, _DIM), jnp.float32),
        scratch_types=[
            pltpu.VMEM((rows_per_w, _LANES), jnp.int32),
            pltpu.VMEM((_LANES, _DIM), jnp.float32),
            pltpu.SemaphoreType.DMA,
        ],
        compiler_params=pltpu.CompilerParams(use_tc_tiling_on_sc=False),
    )(functools.partial(_gather_kernel, rows_per_w=rows_per_w))

    out = run(ids3d, weight)
    return out.reshape(b, s, _DIM)
